# baseline (speedup 1.0000x reference)
import functools

import jax
import jax.numpy as jnp
from jax import lax
from jax.experimental import pallas as pl
from jax.experimental.pallas import tpu as pltpu
from jax.experimental.pallas import tpu_sc as plsc

N_TC_WORKERS = 32


def _mm_body(x_ref, w_ref, o_ref):
    o_ref[...] = jnp.dot(x_ref[...], w_ref[...],
                         preferred_element_type=jnp.float32)


def _matmul(x, w, blk):
    n, kdim = x.shape
    m = w.shape[1]
    return pl.pallas_call(
        _mm_body,
        grid=(n // blk,),
        in_specs=[
            pl.BlockSpec((blk, kdim), lambda i: (i, 0)),
            pl.BlockSpec((kdim, m), lambda i: (0, 0)),
        ],
        out_specs=pl.BlockSpec((blk, m), lambda i: (i, 0)),
        out_shape=jax.ShapeDtypeStruct((n, m), jnp.float32),
    )(x, w)


def _sc_gather(table, idx):
    bsz = idx.shape[0]
    d = table.shape[1]
    per_w = bsz // N_TC_WORKERS
    chunk = 80
    n_chunks = per_w // chunk
    assert per_w % chunk == 0
    mesh = plsc.VectorSubcoreMesh(core_axis_name="c", subcore_axis_name="s")

    @functools.partial(
        pl.kernel, mesh=mesh,
        out_type=jax.ShapeDtypeStruct((bsz, d), jnp.float32),
        scratch_types=[
            pltpu.VMEM((chunk,), jnp.int32),
            pltpu.VMEM((chunk, d), jnp.float32),
            pltpu.SemaphoreType.DMA,
        ],
    )
    def gather_k(table_hbm, idx_hbm, out_hbm, idx_v, rows_v, sem):
        wid = lax.axis_index("s") * 2 + lax.axis_index("c")
        base = wid * per_w

        def body(i, _):
            off = base + i * chunk
            pltpu.sync_copy(idx_hbm.at[pl.ds(off, chunk)], idx_v)
            pltpu.async_copy(table_hbm.at[idx_v], rows_v, sem).wait()
            pltpu.sync_copy(rows_v, out_hbm.at[pl.ds(off, chunk)])
            return 0

        lax.fori_loop(0, n_chunks, body, 0)

    return gather_k(table, idx)


_E_CHUNK = 2048
_BATCH = 128
_PEND = 2304
_Q = 1280
_REG = 2304


def _sc_scatter_add(src_pad, dst_pad, h0, n_rows):
    d = h0.shape[1]
    e_pad = src_pad.shape[0]
    per_tile_e = e_pad // 16
    n_chunks = per_tile_e // _E_CHUNK
    assert per_tile_e % _E_CHUNK == 0
    half = n_rows // 2
    rng = 10000
    n_pass = half // rng
    assert half % n_pass == 0
    rows_per_tile = 624
    base_pieces = [(0, 128), (128, 128), (256, 128), (384, 128), (512, 112)]
    extra_piece = (15 * 624, 16)
    mesh = plsc.VectorSubcoreMesh(core_axis_name="c", subcore_axis_name="s")

    @functools.partial(
        pl.kernel, mesh=mesh,
        out_type=jax.ShapeDtypeStruct((n_rows, d), jnp.float32),
        scratch_types=[
            pltpu.VMEM((_E_CHUNK,), jnp.int32),
            pltpu.VMEM((_E_CHUNK,), jnp.int32),
            pltpu.VMEM((_PEND,), jnp.int32),
            pltpu.VMEM((_PEND,), jnp.int32),
            pltpu.VMEM((_Q,), jnp.int32),
            pltpu.VMEM((_Q,), jnp.int32),
            pltpu.VMEM((_BATCH,), jnp.int32),
        ] + [pltpu.VMEM((16,), jnp.int32) for _ in range(8)] + [
            pltpu.VMEM((16,), jnp.int32),
            pltpu.VMEM((256,), jnp.int32),
            pltpu.VMEM((256,), jnp.int32),
            pltpu.VMEM((256,), jnp.int32),
            pltpu.VMEM((_BATCH, 128), jnp.float32),
            pltpu.VMEM((656, 128), jnp.float32),
            pltpu.VMEM_SHARED((16 * _REG,), jnp.int32),
            pltpu.VMEM_SHARED((16 * _REG,), jnp.int32),
            pltpu.VMEM_SHARED((256,), jnp.int32),
            pltpu.SemaphoreType.DMA,
        ],
    )
    def scatter_k(src_hbm, dst_hbm, h0_hbm, zeros_hbm, agg_hbm,
                  src_v, dst_v, pend_src, pend_dst, q_src, q_dst, bat_src,
                  b16_0, b16_1, b16_2, b16_3, b16_4, b16_5, b16_6, b16_7,
                  cnt_pub, sbuf_src, sbuf_dst, sbuf_cnt, rows_v, acc,
                  stage_src, stage_dst, stage_cnt, sem):
        bat16 = [b16_0, b16_1, b16_2, b16_3, b16_4, b16_5, b16_6, b16_7]
        c = lax.axis_index("c")
        s = lax.axis_index("s")

        lane = lax.iota(jnp.int32, 16)
        one = jnp.full((16,), 1, jnp.int32)
        zero16 = jnp.full((16,), 0, jnp.int32)
        rng_u = jnp.full((16,), rng, jnp.uint32)
        dummy_v = jnp.full((16,), rng, jnp.int32)
        lane_u = lane.astype(jnp.uint32)
        pshift = [(jnp.maximum(lane - kk, 0),
                   lane_u >= jnp.full((16,), kk, jnp.uint32))
                  for kk in (1, 2, 4, 8)]
        rot_idx = [jnp.maximum(lane - r, 0) for r in range(1, 16)]
        tgt = lane + one
        own_lo = s * rows_per_tile
        own_sz = jnp.where(s == 15, 640, 624)
        own_lo_v = jnp.full((16,), own_lo, jnp.int32)
        own_sz_u = jnp.full((16,), own_sz, jnp.int32).astype(jnp.uint32)

        def _id(x):
            return x

        def init_pend(i, _):
            pend_src[pl.ds(i * 16, 16)] = zero16
            return 0
        lax.fori_loop(0, _PEND // 16, init_pend, 0)

        def init_q(i, _):
            q_src[pl.ds(i * 16, 16)] = zero16
            return 0
        lax.fori_loop(0, _Q // 16, init_q, 0)

        def append_compact(dref, sref, dvx, svx, mind, cur):
            p = mind
            for idxk, mk in pshift:
                p = p + jnp.where(mk, jnp.take(p, idxk), zero16)
            cnt = p[15]

            @pl.when(cnt > 0)
            def _():
                pos = zero16
                for kk in (8, 4, 2, 1):
                    npos = pos + jnp.full((16,), kk, jnp.int32)
                    pv = jnp.take(p, npos - one)
                    pos = jnp.where(pv < tgt, npos, pos)
                dref[pl.ds(cur, 16)] = jnp.take(dvx, pos)
                sref[pl.ds(cur, 16)] = jnp.take(svx, pos)
            return cur + cnt

        def drain_batch(cur):
            cur = cur - _BATCH
            for j in range(8):
                bat_src[pl.ds(j * 16, 16)] = q_src[pl.ds(cur + j * 16, 16)]
            pltpu.async_copy(h0_hbm.at[bat_src], rows_v, sem).wait()

            def addrow(j, _):
                dvb = q_dst[pl.ds(cur + j * 16, 16)]
                for l in range(16):
                    dloc = dvb[l]
                    for u in range(8):
                        acc[dloc, pl.ds(u * 16, 16)] = (
                            acc[dloc, pl.ds(u * 16, 16)]
                            + rows_v[j * 16 + l, pl.ds(u * 16, 16)])
                return 0
            lax.fori_loop(0, 8, addrow, 0)
            return cur

        qdummy_v = jnp.full((16,), 648, jnp.int32)

        def drain_all(cur):
            def pad_step(_, cu):
                def do_pad(c2):
                    q_dst[pl.ds(c2, 16)] = qdummy_v
                    return c2 + jnp.minimum(16, _BATCH - (c2 & (_BATCH - 1)))
                return lax.cond((cu & (_BATCH - 1)) != 0, do_pad, _id, cu)

            def round_(_, cu):
                def do_round(c2):
                    c2 = lax.fori_loop(0, 8, pad_step, c2)
                    return drain_batch(c2)
                return lax.cond(cu > 0, do_round, _id, cu)

            return lax.fori_loop(0, 4, round_, cur)

        def do_pass(p, _):
            lo = c * half + p * rng
            for zoff in (0, 128, 256, 384, 512):
                pltpu.sync_copy(zeros_hbm, acc.at[pl.ds(zoff, 128)])
            pltpu.sync_copy(zeros_hbm.at[pl.ds(0, 16)], acc.at[pl.ds(640, 16)])
            plsc.subcore_barrier()

            lo_v = jnp.full((16,), lo, jnp.int32)

            def do_chunk(ci, qcur):
                ebase = s * per_tile_e + ci * _E_CHUNK
                pltpu.sync_copy(src_hbm.at[pl.ds(ebase, _E_CHUNK)], src_v)
                pltpu.sync_copy(dst_hbm.at[pl.ds(ebase, _E_CHUNK)], dst_v)

                def scan(i, cur):
                    dv = dst_v[pl.ds(i * 16, 16)]
                    sv = src_v[pl.ds(i * 16, 16)]
                    m = (dv - lo_v).astype(jnp.uint32) < rng_u
                    mi = jnp.where(m, one, zero16)
                    return append_compact(pend_dst, pend_src,
                                          dv - lo_v, sv, mi, cur)

                cur = lax.fori_loop(0, _E_CHUNK // 16, scan, 0)

                def padp(c2):
                    pend_dst[pl.ds(c2, 16)] = dummy_v
                    return c2 + (16 - (c2 & 15))
                cur = lax.cond((cur & 15) != 0, padp, _id, cur)

                cnt_pub[...] = jnp.full((16,), cur, jnp.int32)
                pltpu.sync_copy(cnt_pub, stage_cnt.at[pl.ds(s * 16, 16)])
                nb = (cur + 255) // 256

                def cpb(b, _):
                    pltpu.sync_copy(
                        pend_src.at[pl.ds(b * 256, 256)],
                        stage_src.at[pl.ds(s * _REG + b * 256, 256)])
                    pltpu.sync_copy(
                        pend_dst.at[pl.ds(b * 256, 256)],
                        stage_dst.at[pl.ds(s * _REG + b * 256, 256)])
                    return 0
                lax.fori_loop(0, nb, cpb, 0)
                plsc.subcore_barrier()

                pltpu.sync_copy(stage_cnt, sbuf_cnt)

                def per_prod(pp, qcur):
                    cnt_p = sbuf_cnt[pl.ds(pp * 16, 16)][0]
                    nb2 = (cnt_p + 255) // 256

                    def per_blk(b, qcur):
                        pltpu.sync_copy(
                            stage_src.at[pl.ds(pp * _REG + b * 256, 256)],
                            sbuf_src)
                        pltpu.sync_copy(
                            stage_dst.at[pl.ds(pp * _REG + b * 256, 256)],
                            sbuf_dst)
                        cnt_u = jnp.full((16,), cnt_p,
                                         jnp.int32).astype(jnp.uint32)

                        def per_vreg(v, qcur):
                            dv = sbuf_dst[pl.ds(v * 16, 16)]
                            sv = sbuf_src[pl.ds(v * 16, 16)]
                            gidx = jnp.full((16,), b * 256 + v * 16,
                                            jnp.int32) + lane
                            okv = gidx.astype(jnp.uint32) < cnt_u
                            mine = ((dv - own_lo_v).astype(jnp.uint32)
                                    < own_sz_u)
                            ind = jnp.where(mine,
                                            jnp.where(okv, one, zero16),
                                            zero16)
                            qcur = append_compact(q_dst, q_src,
                                                  dv - own_lo_v, sv, ind, qcur)
                            qcur = lax.cond(qcur >= 192, drain_batch,
                                            _id, qcur)
                            qcur = lax.cond(qcur >= 192, drain_batch,
                                            _id, qcur)
                            return qcur

                        return lax.fori_loop(0, 16, per_vreg, qcur)

                    return lax.fori_loop(0, nb2, per_blk, qcur)

                qcur = lax.fori_loop(0, 16, per_prod, qcur)
                plsc.subcore_barrier()
                return qcur

            qcur = lax.fori_loop(0, n_chunks, do_chunk, 0)
            drain_all(qcur)
            for off, sz in base_pieces:
                pltpu.sync_copy(acc.at[pl.ds(off, sz)],
                                agg_hbm.at[pl.ds(lo + own_lo + off, sz)])

            @pl.when(s == 15)
            def _():
                pltpu.sync_copy(acc.at[pl.ds(624, 16)],
                                agg_hbm.at[pl.ds(lo + 9984, 16)])
            return 0

        lax.fori_loop(0, n_pass, do_pass, 0)

    zeros_in = jnp.zeros((128, 128), jnp.float32)
    return scatter_k(src_pad, dst_pad, h0, zeros_in)


def _mlp_pool_body(ksub, h0_ref, agg_ref, w1_ref, b1_ref, w2_ref, b2_ref, o_ref):
    h = h0_ref[...] + agg_ref[...]
    h = jnp.maximum(jnp.dot(h, w1_ref[...], preferred_element_type=jnp.float32)
                    + b1_ref[...], 0.0)
    h = jnp.maximum(jnp.dot(h, w2_ref[...], preferred_element_type=jnp.float32)
                    + b2_ref[...], 0.0)
    nsub = h.shape[0] // ksub
    h3 = h.reshape(nsub, ksub, h.shape[1])
    k2 = ksub
    while k2 > 1:
        hlf = k2 // 2
        h3 = h3[:, :hlf, :] + h3[:, hlf:2 * hlf, :] if k2 % 2 == 0 else (
            jnp.concatenate([h3[:, :hlf, :] + h3[:, hlf:2 * hlf, :],
                             h3[:, 2 * hlf:, :]], axis=1))
        k2 = k2 - hlf
    o_ref[...] = h3.reshape(nsub, h.shape[1]) * (1.0 / ksub)


def _mlp_pool(h0, agg, w1, b1, w2, b2, ksub):
    bk, hdim = h0.shape
    blk = 12800
    nsub_blk = blk // ksub
    return pl.pallas_call(
        functools.partial(_mlp_pool_body, ksub),
        grid=(bk // blk,),
        in_specs=[
            pl.BlockSpec((blk, hdim), lambda i: (i, 0)),
            pl.BlockSpec((blk, hdim), lambda i: (i, 0)),
            pl.BlockSpec((hdim, hdim), lambda i: (0, 0)),
            pl.BlockSpec((1, hdim), lambda i: (0, 0)),
            pl.BlockSpec((hdim, hdim), lambda i: (0, 0)),
            pl.BlockSpec((1, hdim), lambda i: (0, 0)),
        ],
        out_specs=pl.BlockSpec((nsub_blk, hdim), lambda i: (i, 0)),
        out_shape=jax.ShapeDtypeStruct((bk // ksub, hdim), jnp.float32),
    )(h0, agg, w1, b1.reshape(1, -1), w2, b2.reshape(1, -1))


def _head_body(num_classes, gid_ref, reps_ref, wh1_ref, bh1_ref,
               wh2_ref, bh2_ref, logits_ref, probs_ref, preds_ref,
               onehot_ref, acc, cnt):
    i = pl.program_id(0)
    ngraphs = acc.shape[0]
    nblk = reps_ref.shape[0]

    @pl.when(i == 0)
    def _():
        acc[...] = jnp.zeros_like(acc)
        cnt[...] = jnp.zeros_like(cnt)

    gid = gid_ref[0, 0, :]
    gio = jax.lax.broadcasted_iota(jnp.int32, (ngraphs, nblk), 0)
    q = (gid[None, :] == gio).astype(jnp.float32)
    acc[...] += jnp.dot(q, reps_ref[...], preferred_element_type=jnp.float32,
                        precision=lax.Precision.HIGHEST)
    cnt[...] += jnp.broadcast_to(jnp.sum(q, axis=1, keepdims=True),
                                 cnt.shape)

    @pl.when(i == pl.num_programs(0) - 1)
    def _():
        per_graph = acc[...] / jnp.maximum(cnt[...], 1.0)
        t = jnp.maximum(
            jnp.dot(per_graph, wh1_ref[...], preferred_element_type=jnp.float32)
            + bh1_ref[...], 0.0)
        lg = jnp.dot(t, wh2_ref[...], preferred_element_type=jnp.float32) \
            + bh2_ref[...]
        col = jax.lax.broadcasted_iota(jnp.int32, lg.shape, 1)
        valid = col < num_classes
        lm = jnp.max(jnp.where(valid, lg, -jnp.inf), axis=1, keepdims=True)
        e = jnp.where(valid, jnp.exp(lg - lm), 0.0)
        probs = e / jnp.sum(e, axis=1, keepdims=True)
        pred = jnp.min(jnp.where((lg == lm) & valid, col, lg.shape[1]),
                       axis=1, keepdims=True)
        logits_ref[...] = lg
        probs_ref[...] = probs
        preds_ref[...] = jnp.broadcast_to(pred, preds_ref.shape)
        onehot_ref[...] = (col == pred).astype(jnp.float32)


def _head(graph_id, reps, wh1, bh1, wh2, bh2, ngraphs, num_classes):
    b, hdim = reps.shape
    blk = 2000
    nsteps = b // blk
    gid3 = graph_id.reshape(nsteps, 1, blk)
    wh2p = jnp.pad(wh2, ((0, 0), (0, hdim - num_classes)))
    bh2p = jnp.pad(bh2, (0, hdim - num_classes)).reshape(1, -1)
    out_sds = jax.ShapeDtypeStruct((ngraphs, hdim), jnp.float32)
    return pl.pallas_call(
        functools.partial(_head_body, num_classes),
        grid=(nsteps,),
        in_specs=[
            pl.BlockSpec((1, 1, blk), lambda i: (i, 0, 0)),
            pl.BlockSpec((blk, hdim), lambda i: (i, 0)),
            pl.BlockSpec((hdim, hdim), lambda i: (0, 0)),
            pl.BlockSpec((1, hdim), lambda i: (0, 0)),
            pl.BlockSpec((hdim, hdim), lambda i: (0, 0)),
            pl.BlockSpec((1, hdim), lambda i: (0, 0)),
        ],
        out_specs=[
            pl.BlockSpec((ngraphs, hdim), lambda i: (0, 0)),
            pl.BlockSpec((ngraphs, hdim), lambda i: (0, 0)),
            pl.BlockSpec((ngraphs, hdim), lambda i: (0, 0)),
            pl.BlockSpec((ngraphs, hdim), lambda i: (0, 0)),
        ],
        out_shape=[out_sds, out_sds,
                   jax.ShapeDtypeStruct((ngraphs, hdim), jnp.int32), out_sds],
        scratch_shapes=[pltpu.VMEM((ngraphs, hdim), jnp.float32),
                        pltpu.VMEM((ngraphs, hdim), jnp.float32)],
    )(gid3, reps, wh1, bh1.reshape(1, -1), wh2p, bh2p)


def kernel(x_global, nodes_t, edge_index_t, edge_ptr_t, graph_id_t, k,
           W_in, W1, b1, W2, b2, Wh1, bh1, Wh2, bh2):
    b, ksub = nodes_t.shape
    bk = b * ksub
    num_classes = Wh2.shape[1]
    ngraphs = 512
    e_total = edge_index_t.shape[1]

    xw = _matmul(x_global, W_in, 1000)
    h0 = _sc_gather(xw, nodes_t.reshape(-1))

    e_pad = ((e_total + 16 * _E_CHUNK - 1) // (16 * _E_CHUNK)) * (16 * _E_CHUNK)
    src_p = jnp.concatenate(
        [edge_index_t[0], jnp.zeros((e_pad - e_total,), jnp.int32)])
    dst_p = jnp.concatenate(
        [edge_index_t[1], jnp.full((e_pad - e_total,), bk, jnp.int32)])
    agg = _sc_scatter_add(src_p, dst_p, h0, bk)

    reps = _mlp_pool(h0, agg, W1, b1, W2, b2, ksub)

    logits_p, probs_p, preds_p, onehot_p = _head(
        graph_id_t, reps, Wh1, bh1, Wh2, bh2, ngraphs, num_classes)
    logits = logits_p[:, :num_classes]
    probs = probs_p[:, :num_classes]
    preds = preds_p[:, 0]
    one_hot = onehot_p[:, :num_classes]
    return (logits, probs, preds, one_hot)


# speedup vs baseline: 1.0763x; 1.0763x over previous
import functools

import jax
import jax.numpy as jnp
from jax import lax
from jax.experimental import pallas as pl
from jax.experimental.pallas import tpu as pltpu
from jax.experimental.pallas import tpu_sc as plsc

N_TC_WORKERS = 32


def _mm_body(x_ref, w_ref, o_ref):
    o_ref[...] = jnp.dot(x_ref[...], w_ref[...],
                         preferred_element_type=jnp.float32)


def _matmul(x, w, blk):
    n, kdim = x.shape
    m = w.shape[1]
    return pl.pallas_call(
        _mm_body,
        grid=(n // blk,),
        in_specs=[
            pl.BlockSpec((blk, kdim), lambda i: (i, 0)),
            pl.BlockSpec((kdim, m), lambda i: (0, 0)),
        ],
        out_specs=pl.BlockSpec((blk, m), lambda i: (i, 0)),
        out_shape=jax.ShapeDtypeStruct((n, m), jnp.float32),
    )(x, w)


def _sc_gather(table, idx):
    bsz = idx.shape[0]
    d = table.shape[1]
    per_w = bsz // N_TC_WORKERS
    chunk = 80
    n_chunks = per_w // chunk
    assert per_w % chunk == 0
    mesh = plsc.VectorSubcoreMesh(core_axis_name="c", subcore_axis_name="s")

    @functools.partial(
        pl.kernel, mesh=mesh,
        out_type=jax.ShapeDtypeStruct((bsz, d), jnp.float32),
        scratch_types=[
            pltpu.VMEM((chunk,), jnp.int32),
            pltpu.VMEM((chunk, d), jnp.float32),
            pltpu.SemaphoreType.DMA,
        ],
    )
    def gather_k(table_hbm, idx_hbm, out_hbm, idx_v, rows_v, sem):
        wid = lax.axis_index("s") * 2 + lax.axis_index("c")
        base = wid * per_w

        def body(i, _):
            off = base + i * chunk
            pltpu.sync_copy(idx_hbm.at[pl.ds(off, chunk)], idx_v)
            pltpu.async_copy(table_hbm.at[idx_v], rows_v, sem).wait()
            pltpu.sync_copy(rows_v, out_hbm.at[pl.ds(off, chunk)])
            return 0

        lax.fori_loop(0, n_chunks, body, 0)

    return gather_k(table, idx)


_E_CHUNK = 2048
_BATCH = 128
_PEND = 2304
_Q = 1280
_REG = 2304


def _sc_scatter_add(src_pad, dst_pad, h0, n_rows):
    d = h0.shape[1]
    e_pad = src_pad.shape[0]
    per_tile_e = e_pad // 16
    n_chunks = per_tile_e // _E_CHUNK
    assert per_tile_e % _E_CHUNK == 0
    half = n_rows // 2
    rng = 10000
    n_pass = half // rng
    assert half % n_pass == 0
    rows_per_tile = 624
    base_pieces = [(0, 128), (128, 128), (256, 128), (384, 128), (512, 112)]
    extra_piece = (15 * 624, 16)
    mesh = plsc.VectorSubcoreMesh(core_axis_name="c", subcore_axis_name="s")

    @functools.partial(
        pl.kernel, mesh=mesh,
        out_type=jax.ShapeDtypeStruct((n_rows, d), jnp.float32),
        scratch_types=[
            pltpu.VMEM((_E_CHUNK,), jnp.int32),
            pltpu.VMEM((_E_CHUNK,), jnp.int32),
            pltpu.VMEM((_PEND,), jnp.int32),
            pltpu.VMEM((_PEND,), jnp.int32),
            pltpu.VMEM((_Q,), jnp.int32),
            pltpu.VMEM((_Q,), jnp.int32),
            pltpu.VMEM((_BATCH,), jnp.int32),
        ] + [pltpu.VMEM((16,), jnp.int32) for _ in range(8)] + [
            pltpu.VMEM((16,), jnp.int32),
            pltpu.VMEM((4096,), jnp.int32),
            pltpu.VMEM((4096,), jnp.int32),
            pltpu.VMEM((256,), jnp.int32),
            pltpu.VMEM((256,), jnp.int32),
            pltpu.VMEM((256,), jnp.int32),
            pltpu.VMEM((_BATCH, 128), jnp.float32),
            pltpu.VMEM((656, 128), jnp.float32),
            pltpu.VMEM_SHARED((16 * _REG,), jnp.int32),
            pltpu.VMEM_SHARED((16 * _REG,), jnp.int32),
            pltpu.VMEM_SHARED((256,), jnp.int32),
            pltpu.SemaphoreType.DMA,
        ],
    )
    def scatter_k(src_hbm, dst_hbm, h0_hbm, zeros_hbm, agg_hbm,
                  src_v, dst_v, pend_src, pend_dst, q_src, q_dst, bat_src,
                  b16_0, b16_1, b16_2, b16_3, b16_4, b16_5, b16_6, b16_7,
                  cnt_pub, sbuf_src, sbuf_dst, sbuf_cnt, ovf_src, ovf_dst,
                  rows_v, acc, stage_src, stage_dst, stage_cnt, sem):
        bat16 = [b16_0, b16_1, b16_2, b16_3, b16_4, b16_5, b16_6, b16_7]
        c = lax.axis_index("c")
        s = lax.axis_index("s")

        lane = lax.iota(jnp.int32, 16)
        one = jnp.full((16,), 1, jnp.int32)
        zero16 = jnp.full((16,), 0, jnp.int32)
        rng_u = jnp.full((16,), rng, jnp.uint32)
        dummy_v = jnp.full((16,), rng, jnp.int32)
        lane_u = lane.astype(jnp.uint32)
        pshift = [(jnp.maximum(lane - kk, 0),
                   lane_u >= jnp.full((16,), kk, jnp.uint32))
                  for kk in (1, 2, 4, 8)]
        rot_idx = [jnp.maximum(lane - r, 0) for r in range(1, 16)]
        tgt = lane + one
        own_lo = s * rows_per_tile
        own_sz = jnp.where(s == 15, 640, 624)
        own_lo_v = jnp.full((16,), own_lo, jnp.int32)
        own_sz_u = jnp.full((16,), own_sz, jnp.int32).astype(jnp.uint32)

        def _id(x):
            return x

        def init_pend(i, _):
            pend_src[pl.ds(i * 16, 16)] = zero16
            return 0
        lax.fori_loop(0, _PEND // 16, init_pend, 0)

        def init_q(i, _):
            q_src[pl.ds(i * 16, 16)] = zero16
            return 0
        lax.fori_loop(0, _Q // 16, init_q, 0)

        def append_compact(dref, sref, dvx, svx, mind, cur):
            p = mind
            for idxk, mk in pshift:
                p = p + jnp.where(mk, jnp.take(p, idxk), zero16)
            cnt = p[15]

            @pl.when(cnt > 0)
            def _():
                pos = zero16
                for kk in (8, 4, 2, 1):
                    npos = pos + jnp.full((16,), kk, jnp.int32)
                    pv = jnp.take(p, npos - one)
                    pos = jnp.where(pv < tgt, npos, pos)
                dref[pl.ds(cur, 16)] = jnp.take(dvx, pos)
                sref[pl.ds(cur, 16)] = jnp.take(svx, pos)
            return cur + cnt

        def drain_batch(cur):
            cur = cur - _BATCH
            for j in range(8):
                bat_src[pl.ds(j * 16, 16)] = q_src[pl.ds(cur + j * 16, 16)]
            pltpu.async_copy(h0_hbm.at[bat_src], rows_v, sem).wait()

            def addrow(j, _):
                dvb = q_dst[pl.ds(cur + j * 16, 16)]
                for l in range(16):
                    dloc = dvb[l]
                    for u in range(8):
                        acc[dloc, pl.ds(u * 16, 16)] = (
                            acc[dloc, pl.ds(u * 16, 16)]
                            + rows_v[j * 16 + l, pl.ds(u * 16, 16)])
                return 0
            lax.fori_loop(0, 8, addrow, 0)
            return cur

        qdummy_v = jnp.full((16,), 648, jnp.int32)

        def drain_all(cur):
            def pad_step(_, cu):
                def do_pad(c2):
                    q_dst[pl.ds(c2, 16)] = qdummy_v
                    return c2 + jnp.minimum(16, _BATCH - (c2 & (_BATCH - 1)))
                return lax.cond((cu & (_BATCH - 1)) != 0, do_pad, _id, cu)

            def round_(_, cu):
                def do_round(c2):
                    c2 = lax.fori_loop(0, 8, pad_step, c2)
                    return drain_batch(c2)
                return lax.cond(cu > 0, do_round, _id, cu)

            return lax.fori_loop(0, 4, round_, cur)

        def do_pass(p, _):
            lo = c * half + p * rng
            for zoff in (0, 128, 256, 384, 512):
                pltpu.sync_copy(zeros_hbm, acc.at[pl.ds(zoff, 128)])
            pltpu.sync_copy(zeros_hbm.at[pl.ds(0, 16)], acc.at[pl.ds(640, 16)])
            plsc.subcore_barrier()

            lo_v = jnp.full((16,), lo, jnp.int32)

            def do_chunk(ci, qcur):
                ebase = s * per_tile_e + ci * _E_CHUNK
                pltpu.sync_copy(src_hbm.at[pl.ds(ebase, _E_CHUNK)], src_v)
                pltpu.sync_copy(dst_hbm.at[pl.ds(ebase, _E_CHUNK)], dst_v)

                def scan(i, cur):
                    dv = dst_v[pl.ds(i * 16, 16)]
                    sv = src_v[pl.ds(i * 16, 16)]
                    m = (dv - lo_v).astype(jnp.uint32) < rng_u
                    mi = jnp.where(m, one, zero16)
                    return append_compact(pend_dst, pend_src,
                                          dv - lo_v, sv, mi, cur)

                cur = lax.fori_loop(0, _E_CHUNK // 16, scan, 0)

                def padp(c2):
                    pend_dst[pl.ds(c2, 16)] = dummy_v
                    return c2 + (16 - (c2 & 15))
                cur = lax.cond((cur & 15) != 0, padp, _id, cur)

                cnt_pub[...] = jnp.full((16,), cur, jnp.int32)
                pltpu.sync_copy(cnt_pub, stage_cnt.at[pl.ds(s * 16, 16)])
                nb = (cur + 255) // 256

                def cpb(b, _):
                    pltpu.sync_copy(
                        pend_src.at[pl.ds(b * 256, 256)],
                        stage_src.at[pl.ds(s * _REG + b * 256, 256)])
                    pltpu.sync_copy(
                        pend_dst.at[pl.ds(b * 256, 256)],
                        stage_dst.at[pl.ds(s * _REG + b * 256, 256)])
                    return 0
                lax.fori_loop(0, nb, cpb, 0)
                plsc.subcore_barrier()

                pltpu.sync_copy(stage_cnt, sbuf_cnt)
                cps = []
                for pp in range(16):
                    cps.append(pltpu.async_copy(
                        stage_src.at[pl.ds(pp * _REG, 256)],
                        sbuf_src.at[pl.ds(pp * 256, 256)], sem))
                    cps.append(pltpu.async_copy(
                        stage_dst.at[pl.ds(pp * _REG, 256)],
                        sbuf_dst.at[pl.ds(pp * 256, 256)], sem))
                for cp in cps:
                    cp.wait()

                def scan_blk(dref, sref, boff, cnt_p, base_g, qcur):
                    cnt_u = jnp.full((16,), cnt_p, jnp.int32).astype(jnp.uint32)

                    def per_vreg(v, qcur):
                        dv = dref[pl.ds(boff + v * 16, 16)]
                        sv = sref[pl.ds(boff + v * 16, 16)]
                        gidx = jnp.full((16,), base_g, jnp.int32) \
                            + jnp.full((16,), v * 16, jnp.int32) + lane
                        okv = gidx.astype(jnp.uint32) < cnt_u
                        mine = (dv - own_lo_v).astype(jnp.uint32) < own_sz_u
                        ind = jnp.where(mine, jnp.where(okv, one, zero16),
                                        zero16)
                        qcur = append_compact(q_dst, q_src,
                                              dv - own_lo_v, sv, ind, qcur)
                        qcur = lax.cond(qcur >= 192, drain_batch, _id, qcur)
                        qcur = lax.cond(qcur >= 192, drain_batch, _id, qcur)
                        return qcur

                    return lax.fori_loop(0, 16, per_vreg, qcur)

                def per_prod(pp, qcur):
                    cnt_p = sbuf_cnt[pl.ds(pp * 16, 16)][0]
                    qcur = scan_blk(sbuf_dst, sbuf_src, pp * 256, cnt_p,
                                    0, qcur)
                    nb2 = (cnt_p + 255) // 256

                    def per_blk(b, qcur):
                        pltpu.sync_copy(
                            stage_src.at[pl.ds(pp * _REG + b * 256, 256)],
                            ovf_src)
                        pltpu.sync_copy(
                            stage_dst.at[pl.ds(pp * _REG + b * 256, 256)],
                            ovf_dst)
                        return scan_blk(ovf_dst, ovf_src, 0, cnt_p,
                                        b * 256, qcur)

                    return lax.cond(
                        nb2 > 1,
                        lambda qc: lax.fori_loop(1, nb2, per_blk, qc),
                        _id, qcur)

                qcur = lax.fori_loop(0, 16, per_prod, qcur)
                plsc.subcore_barrier()
                return qcur

            qcur = lax.fori_loop(0, n_chunks, do_chunk, 0)
            drain_all(qcur)
            for off, sz in base_pieces:
                pltpu.sync_copy(acc.at[pl.ds(off, sz)],
                                agg_hbm.at[pl.ds(lo + own_lo + off, sz)])

            @pl.when(s == 15)
            def _():
                pltpu.sync_copy(acc.at[pl.ds(624, 16)],
                                agg_hbm.at[pl.ds(lo + 9984, 16)])
            return 0

        lax.fori_loop(0, n_pass, do_pass, 0)

    zeros_in = jnp.zeros((128, 128), jnp.float32)
    return scatter_k(src_pad, dst_pad, h0, zeros_in)


def _mlp_pool_body(ksub, h0_ref, agg_ref, w1_ref, b1_ref, w2_ref, b2_ref, o_ref):
    h = h0_ref[...] + agg_ref[...]
    h = jnp.maximum(jnp.dot(h, w1_ref[...], preferred_element_type=jnp.float32)
                    + b1_ref[...], 0.0)
    h = jnp.maximum(jnp.dot(h, w2_ref[...], preferred_element_type=jnp.float32)
                    + b2_ref[...], 0.0)
    nsub = h.shape[0] // ksub
    h3 = h.reshape(nsub, ksub, h.shape[1])
    k2 = ksub
    while k2 > 1:
        hlf = k2 // 2
        h3 = h3[:, :hlf, :] + h3[:, hlf:2 * hlf, :] if k2 % 2 == 0 else (
            jnp.concatenate([h3[:, :hlf, :] + h3[:, hlf:2 * hlf, :],
                             h3[:, 2 * hlf:, :]], axis=1))
        k2 = k2 - hlf
    o_ref[...] = h3.reshape(nsub, h.shape[1]) * (1.0 / ksub)


def _mlp_pool(h0, agg, w1, b1, w2, b2, ksub):
    bk, hdim = h0.shape
    blk = 12800
    nsub_blk = blk // ksub
    return pl.pallas_call(
        functools.partial(_mlp_pool_body, ksub),
        grid=(bk // blk,),
        in_specs=[
            pl.BlockSpec((blk, hdim), lambda i: (i, 0)),
            pl.BlockSpec((blk, hdim), lambda i: (i, 0)),
            pl.BlockSpec((hdim, hdim), lambda i: (0, 0)),
            pl.BlockSpec((1, hdim), lambda i: (0, 0)),
            pl.BlockSpec((hdim, hdim), lambda i: (0, 0)),
            pl.BlockSpec((1, hdim), lambda i: (0, 0)),
        ],
        out_specs=pl.BlockSpec((nsub_blk, hdim), lambda i: (i, 0)),
        out_shape=jax.ShapeDtypeStruct((bk // ksub, hdim), jnp.float32),
    )(h0, agg, w1, b1.reshape(1, -1), w2, b2.reshape(1, -1))


def _head_body(num_classes, gid_ref, reps_ref, wh1_ref, bh1_ref,
               wh2_ref, bh2_ref, logits_ref, probs_ref, preds_ref,
               onehot_ref, acc, cnt):
    i = pl.program_id(0)
    ngraphs = acc.shape[0]
    nblk = reps_ref.shape[0]

    @pl.when(i == 0)
    def _():
        acc[...] = jnp.zeros_like(acc)
        cnt[...] = jnp.zeros_like(cnt)

    gid = gid_ref[0, 0, :]
    gio = jax.lax.broadcasted_iota(jnp.int32, (ngraphs, nblk), 0)
    q = (gid[None, :] == gio).astype(jnp.float32)
    acc[...] += jnp.dot(q, reps_ref[...], preferred_element_type=jnp.float32,
                        precision=lax.Precision.HIGHEST)
    cnt[...] += jnp.broadcast_to(jnp.sum(q, axis=1, keepdims=True),
                                 cnt.shape)

    @pl.when(i == pl.num_programs(0) - 1)
    def _():
        per_graph = acc[...] / jnp.maximum(cnt[...], 1.0)
        t = jnp.maximum(
            jnp.dot(per_graph, wh1_ref[...], preferred_element_type=jnp.float32)
            + bh1_ref[...], 0.0)
        lg = jnp.dot(t, wh2_ref[...], preferred_element_type=jnp.float32) \
            + bh2_ref[...]
        col = jax.lax.broadcasted_iota(jnp.int32, lg.shape, 1)
        valid = col < num_classes
        lm = jnp.max(jnp.where(valid, lg, -jnp.inf), axis=1, keepdims=True)
        e = jnp.where(valid, jnp.exp(lg - lm), 0.0)
        probs = e / jnp.sum(e, axis=1, keepdims=True)
        pred = jnp.min(jnp.where((lg == lm) & valid, col, lg.shape[1]),
                       axis=1, keepdims=True)
        logits_ref[...] = lg
        probs_ref[...] = probs
        preds_ref[...] = jnp.broadcast_to(pred, preds_ref.shape)
        onehot_ref[...] = (col == pred).astype(jnp.float32)


def _head(graph_id, reps, wh1, bh1, wh2, bh2, ngraphs, num_classes):
    b, hdim = reps.shape
    blk = 2000
    nsteps = b // blk
    gid3 = graph_id.reshape(nsteps, 1, blk)
    wh2p = jnp.pad(wh2, ((0, 0), (0, hdim - num_classes)))
    bh2p = jnp.pad(bh2, (0, hdim - num_classes)).reshape(1, -1)
    out_sds = jax.ShapeDtypeStruct((ngraphs, hdim), jnp.float32)
    return pl.pallas_call(
        functools.partial(_head_body, num_classes),
        grid=(nsteps,),
        in_specs=[
            pl.BlockSpec((1, 1, blk), lambda i: (i, 0, 0)),
            pl.BlockSpec((blk, hdim), lambda i: (i, 0)),
            pl.BlockSpec((hdim, hdim), lambda i: (0, 0)),
            pl.BlockSpec((1, hdim), lambda i: (0, 0)),
            pl.BlockSpec((hdim, hdim), lambda i: (0, 0)),
            pl.BlockSpec((1, hdim), lambda i: (0, 0)),
        ],
        out_specs=[
            pl.BlockSpec((ngraphs, hdim), lambda i: (0, 0)),
            pl.BlockSpec((ngraphs, hdim), lambda i: (0, 0)),
            pl.BlockSpec((ngraphs, hdim), lambda i: (0, 0)),
            pl.BlockSpec((ngraphs, hdim), lambda i: (0, 0)),
        ],
        out_shape=[out_sds, out_sds,
                   jax.ShapeDtypeStruct((ngraphs, hdim), jnp.int32), out_sds],
        scratch_shapes=[pltpu.VMEM((ngraphs, hdim), jnp.float32),
                        pltpu.VMEM((ngraphs, hdim), jnp.float32)],
    )(gid3, reps, wh1, bh1.reshape(1, -1), wh2p, bh2p)


def kernel(x_global, nodes_t, edge_index_t, edge_ptr_t, graph_id_t, k,
           W_in, W1, b1, W2, b2, Wh1, bh1, Wh2, bh2):
    b, ksub = nodes_t.shape
    bk = b * ksub
    num_classes = Wh2.shape[1]
    ngraphs = 512
    e_total = edge_index_t.shape[1]

    xw = _matmul(x_global, W_in, 1000)
    h0 = _sc_gather(xw, nodes_t.reshape(-1))

    e_pad = ((e_total + 16 * _E_CHUNK - 1) // (16 * _E_CHUNK)) * (16 * _E_CHUNK)
    src_p = jnp.concatenate(
        [edge_index_t[0], jnp.zeros((e_pad - e_total,), jnp.int32)])
    dst_p = jnp.concatenate(
        [edge_index_t[1], jnp.full((e_pad - e_total,), bk, jnp.int32)])
    agg = _sc_scatter_add(src_p, dst_p, h0, bk)

    reps = _mlp_pool(h0, agg, W1, b1, W2, b2, ksub)

    logits_p, probs_p, preds_p, onehot_p = _head(
        graph_id_t, reps, Wh1, bh1, Wh2, bh2, ngraphs, num_classes)
    logits = logits_p[:, :num_classes]
    probs = probs_p[:, :num_classes]
    preds = preds_p[:, 0]
    one_hot = onehot_p[:, :num_classes]
    return (logits, probs, preds, one_hot)


# speedup vs baseline: 1.4927x; 1.3869x over previous
import functools

import jax
import jax.numpy as jnp
from jax import lax
from jax.experimental import pallas as pl
from jax.experimental.pallas import tpu as pltpu
from jax.experimental.pallas import tpu_sc as plsc

N_TC_WORKERS = 32


def _mm_body(x_ref, w_ref, o_ref):
    o_ref[...] = jnp.dot(x_ref[...], w_ref[...],
                         preferred_element_type=jnp.float32)


def _matmul(x, w, blk):
    n, kdim = x.shape
    m = w.shape[1]
    return pl.pallas_call(
        _mm_body,
        grid=(n // blk,),
        in_specs=[
            pl.BlockSpec((blk, kdim), lambda i: (i, 0)),
            pl.BlockSpec((kdim, m), lambda i: (0, 0)),
        ],
        out_specs=pl.BlockSpec((blk, m), lambda i: (i, 0)),
        out_shape=jax.ShapeDtypeStruct((n, m), jnp.float32),
    )(x, w)


def _sc_gather(table, idx):
    bsz = idx.shape[0]
    d = table.shape[1]
    per_w = bsz // N_TC_WORKERS
    chunk = 80
    n_chunks = per_w // chunk
    assert per_w % chunk == 0
    mesh = plsc.VectorSubcoreMesh(core_axis_name="c", subcore_axis_name="s")

    @functools.partial(
        pl.kernel, mesh=mesh,
        out_type=jax.ShapeDtypeStruct((bsz, d), jnp.float32),
        scratch_types=[
            pltpu.VMEM((chunk,), jnp.int32),
            pltpu.VMEM((chunk, d), jnp.float32),
            pltpu.SemaphoreType.DMA,
        ],
    )
    def gather_k(table_hbm, idx_hbm, out_hbm, idx_v, rows_v, sem):
        wid = lax.axis_index("s") * 2 + lax.axis_index("c")
        base = wid * per_w

        def body(i, _):
            off = base + i * chunk
            pltpu.sync_copy(idx_hbm.at[pl.ds(off, chunk)], idx_v)
            pltpu.async_copy(table_hbm.at[idx_v], rows_v, sem).wait()
            pltpu.sync_copy(rows_v, out_hbm.at[pl.ds(off, chunk)])
            return 0

        lax.fori_loop(0, n_chunks, body, 0)

    return gather_k(table, idx)


_E_CHUNK = 2048
_BATCH = 128
_PEND = 2304
_Q = 1280
_REG = 2304


def _sc_scatter_add(src_pad, dst_pad, h0, n_rows):
    d = h0.shape[1]
    e_pad = src_pad.shape[0]
    per_tile_e = e_pad // 16
    n_chunks = per_tile_e // _E_CHUNK
    assert per_tile_e % _E_CHUNK == 0
    half = n_rows // 2
    rng = 10000
    n_pass = half // rng
    assert half % n_pass == 0
    rows_per_tile = 624
    base_pieces = [(0, 128), (128, 128), (256, 128), (384, 128), (512, 112)]
    extra_piece = (15 * 624, 16)
    mesh = plsc.VectorSubcoreMesh(core_axis_name="c", subcore_axis_name="s")

    @functools.partial(
        pl.kernel, mesh=mesh,
        out_type=jax.ShapeDtypeStruct((n_rows, d), jnp.float32),
        scratch_types=[
            pltpu.VMEM((_E_CHUNK,), jnp.int32),
            pltpu.VMEM((_E_CHUNK,), jnp.int32),
            pltpu.VMEM((_PEND,), jnp.int32),
            pltpu.VMEM((_PEND,), jnp.int32),
            pltpu.VMEM((_Q,), jnp.int32),
            pltpu.VMEM((_Q,), jnp.int32),
            pltpu.VMEM((_BATCH,), jnp.int32),
        ] + [pltpu.VMEM((16,), jnp.int32) for _ in range(8)] + [
            pltpu.VMEM((16,), jnp.int32),
            pltpu.VMEM((4096,), jnp.int32),
            pltpu.VMEM((4096,), jnp.int32),
            pltpu.VMEM((256,), jnp.int32),
            pltpu.VMEM((256,), jnp.int32),
            pltpu.VMEM((256,), jnp.int32),
            pltpu.VMEM((_BATCH, 128), jnp.float32),
            pltpu.VMEM((656, 128), jnp.float32),
            pltpu.VMEM_SHARED((16 * _REG,), jnp.int32),
            pltpu.VMEM_SHARED((16 * _REG,), jnp.int32),
            pltpu.VMEM_SHARED((256,), jnp.int32),
            pltpu.SemaphoreType.DMA,
        ],
    )
    def scatter_k(src_hbm, dst_hbm, h0_hbm, zeros_hbm, agg_hbm,
                  src_v, dst_v, pend_src, pend_dst, q_src, q_dst, bat_src,
                  b16_0, b16_1, b16_2, b16_3, b16_4, b16_5, b16_6, b16_7,
                  cnt_pub, sbuf_src, sbuf_dst, sbuf_cnt, ovf_src, ovf_dst,
                  rows_v, acc, stage_src, stage_dst, stage_cnt, sem):
        bat16 = [b16_0, b16_1, b16_2, b16_3, b16_4, b16_5, b16_6, b16_7]
        c = lax.axis_index("c")
        s = lax.axis_index("s")

        lane = lax.iota(jnp.int32, 16)
        one = jnp.full((16,), 1, jnp.int32)
        zero16 = jnp.full((16,), 0, jnp.int32)
        rng_u = jnp.full((16,), rng, jnp.uint32)
        dummy_v = jnp.full((16,), rng, jnp.int32)
        lane_u = lane.astype(jnp.uint32)
        pshift = [(jnp.maximum(lane - kk, 0),
                   lane_u >= jnp.full((16,), kk, jnp.uint32))
                  for kk in (1, 2, 4, 8)]
        rot_idx = [jnp.maximum(lane - r, 0) for r in range(1, 16)]
        tgt = lane + one
        own_lo = s * rows_per_tile
        own_sz = jnp.where(s == 15, 640, 624)
        own_lo_v = jnp.full((16,), own_lo, jnp.int32)
        own_sz_u = jnp.full((16,), own_sz, jnp.int32).astype(jnp.uint32)

        def _id(x):
            return x

        def init_pend(i, _):
            pend_src[pl.ds(i * 16, 16)] = zero16
            return 0
        lax.fori_loop(0, _PEND // 16, init_pend, 0)

        def init_q(i, _):
            q_src[pl.ds(i * 16, 16)] = zero16
            return 0
        lax.fori_loop(0, _Q // 16, init_q, 0)

        def append_compact(dref, sref, dvx, svx, mind, cur):
            p = mind
            for idxk, mk in pshift:
                p = p + jnp.where(mk, jnp.take(p, idxk), zero16)
            cnt = p[15]

            @pl.when(cnt > 0)
            def _():
                pos = zero16
                for kk in (8, 4, 2, 1):
                    npos = pos + jnp.full((16,), kk, jnp.int32)
                    pv = jnp.take(p, npos - one)
                    pos = jnp.where(pv < tgt, npos, pos)
                dref[pl.ds(cur, 16)] = jnp.take(dvx, pos)
                sref[pl.ds(cur, 16)] = jnp.take(svx, pos)
            return cur + cnt

        def drain_batch(cur):
            cur = cur - _BATCH
            for j in range(8):
                bat_src[pl.ds(j * 16, 16)] = q_src[pl.ds(cur + j * 16, 16)]
            pltpu.async_copy(h0_hbm.at[bat_src], rows_v, sem).wait()

            def addrow(j, _):
                dvb = q_dst[pl.ds(cur + j * 16, 16)]
                for l in range(16):
                    dloc = dvb[l]
                    for u in range(8):
                        acc[dloc, pl.ds(u * 16, 16)] = (
                            acc[dloc, pl.ds(u * 16, 16)]
                            + rows_v[j * 16 + l, pl.ds(u * 16, 16)])
                return 0
            lax.fori_loop(0, 8, addrow, 0)
            return cur

        qdummy_v = jnp.full((16,), 648, jnp.int32)

        def drain_all(cur):
            def pad_step(_, cu):
                def do_pad(c2):
                    q_dst[pl.ds(c2, 16)] = qdummy_v
                    return c2 + jnp.minimum(16, _BATCH - (c2 & (_BATCH - 1)))
                return lax.cond((cu & (_BATCH - 1)) != 0, do_pad, _id, cu)

            def round_(_, cu):
                def do_round(c2):
                    c2 = lax.fori_loop(0, 8, pad_step, c2)
                    return drain_batch(c2)
                return lax.cond(cu > 0, do_round, _id, cu)

            return lax.fori_loop(0, 4, round_, cur)

        def do_pass(p, _):
            lo = c * half + p * rng
            for zoff in (0, 128, 256, 384, 512):
                pltpu.sync_copy(zeros_hbm, acc.at[pl.ds(zoff, 128)])
            pltpu.sync_copy(zeros_hbm.at[pl.ds(0, 16)], acc.at[pl.ds(640, 16)])
            plsc.subcore_barrier()

            lo_v = jnp.full((16,), lo, jnp.int32)

            def do_chunk(ci, qcur):
                ebase = s * per_tile_e + ci * _E_CHUNK
                pltpu.sync_copy(src_hbm.at[pl.ds(ebase, _E_CHUNK)], src_v)
                pltpu.sync_copy(dst_hbm.at[pl.ds(ebase, _E_CHUNK)], dst_v)

                def scan(i, cur):
                    dv = dst_v[pl.ds(i * 16, 16)]
                    sv = src_v[pl.ds(i * 16, 16)]
                    m = (dv - lo_v).astype(jnp.uint32) < rng_u
                    mi = jnp.where(m, one, zero16)
                    return append_compact(pend_dst, pend_src,
                                          dv - lo_v, sv, mi, cur)

                cur = lax.fori_loop(0, _E_CHUNK // 16, scan, 0)

                def padp(c2):
                    pend_dst[pl.ds(c2, 16)] = dummy_v
                    return c2 + (16 - (c2 & 15))
                cur = lax.cond((cur & 15) != 0, padp, _id, cur)

                cnt_pub[...] = jnp.full((16,), cur, jnp.int32)
                pltpu.sync_copy(cnt_pub, stage_cnt.at[pl.ds(s * 16, 16)])
                nb = (cur + 255) // 256

                def cpb(b, _):
                    pltpu.sync_copy(
                        pend_src.at[pl.ds(b * 256, 256)],
                        stage_src.at[pl.ds(s * _REG + b * 256, 256)])
                    pltpu.sync_copy(
                        pend_dst.at[pl.ds(b * 256, 256)],
                        stage_dst.at[pl.ds(s * _REG + b * 256, 256)])
                    return 0
                lax.fori_loop(0, nb, cpb, 0)
                plsc.subcore_barrier()

                pltpu.sync_copy(stage_cnt, sbuf_cnt)
                cps = []
                for pp in range(16):
                    cps.append(pltpu.async_copy(
                        stage_src.at[pl.ds(pp * _REG, 256)],
                        sbuf_src.at[pl.ds(pp * 256, 256)], sem))
                    cps.append(pltpu.async_copy(
                        stage_dst.at[pl.ds(pp * _REG, 256)],
                        sbuf_dst.at[pl.ds(pp * 256, 256)], sem))
                for cp in cps:
                    cp.wait()

                def scan_blk(dref, sref, boff, cnt_p, base_g, qcur):
                    cnt_u = jnp.full((16,), cnt_p, jnp.int32).astype(jnp.uint32)
                    inblk = jnp.clip(cnt_p - base_g, 0, 256)
                    nv = (inblk + 15) // 16

                    def per_vreg(v, qcur):
                        dv = dref[pl.ds(boff + v * 16, 16)]
                        sv = sref[pl.ds(boff + v * 16, 16)]
                        gidx = jnp.full((16,), base_g, jnp.int32) \
                            + jnp.full((16,), v * 16, jnp.int32) + lane
                        okv = gidx.astype(jnp.uint32) < cnt_u
                        mine = (dv - own_lo_v).astype(jnp.uint32) < own_sz_u
                        ind = jnp.where(mine, jnp.where(okv, one, zero16),
                                        zero16)
                        qcur = append_compact(q_dst, q_src,
                                              dv - own_lo_v, sv, ind, qcur)
                        qcur = lax.cond(qcur >= 192, drain_batch, _id, qcur)
                        qcur = lax.cond(qcur >= 192, drain_batch, _id, qcur)
                        return qcur

                    return lax.fori_loop(0, nv, per_vreg, qcur)

                def per_prod(pp, qcur):
                    cnt_p = sbuf_cnt[pl.ds(pp * 16, 16)][0]
                    qcur = scan_blk(sbuf_dst, sbuf_src, pp * 256, cnt_p,
                                    0, qcur)
                    nb2 = (cnt_p + 255) // 256

                    def per_blk(b, qcur):
                        pltpu.sync_copy(
                            stage_src.at[pl.ds(pp * _REG + b * 256, 256)],
                            ovf_src)
                        pltpu.sync_copy(
                            stage_dst.at[pl.ds(pp * _REG + b * 256, 256)],
                            ovf_dst)
                        return scan_blk(ovf_dst, ovf_src, 0, cnt_p,
                                        b * 256, qcur)

                    return lax.cond(
                        nb2 > 1,
                        lambda qc: lax.fori_loop(1, nb2, per_blk, qc),
                        _id, qcur)

                qcur = lax.fori_loop(0, 16, per_prod, qcur)
                plsc.subcore_barrier()
                return qcur

            qcur = lax.fori_loop(0, n_chunks, do_chunk, 0)
            drain_all(qcur)
            for off, sz in base_pieces:
                pltpu.sync_copy(acc.at[pl.ds(off, sz)],
                                agg_hbm.at[pl.ds(lo + own_lo + off, sz)])

            @pl.when(s == 15)
            def _():
                pltpu.sync_copy(acc.at[pl.ds(624, 16)],
                                agg_hbm.at[pl.ds(lo + 9984, 16)])
            return 0

        lax.fori_loop(0, n_pass, do_pass, 0)

    zeros_in = jnp.zeros((128, 128), jnp.float32)
    return scatter_k(src_pad, dst_pad, h0, zeros_in)


def _mlp_pool_body(ksub, h0_ref, agg_ref, w1_ref, b1_ref, w2_ref, b2_ref, o_ref):
    h = h0_ref[...] + agg_ref[...]
    h = jnp.maximum(jnp.dot(h, w1_ref[...], preferred_element_type=jnp.float32)
                    + b1_ref[...], 0.0)
    h = jnp.maximum(jnp.dot(h, w2_ref[...], preferred_element_type=jnp.float32)
                    + b2_ref[...], 0.0)
    nsub = h.shape[0] // ksub
    h3 = h.reshape(nsub, ksub, h.shape[1])
    k2 = ksub
    while k2 > 1:
        hlf = k2 // 2
        h3 = h3[:, :hlf, :] + h3[:, hlf:2 * hlf, :] if k2 % 2 == 0 else (
            jnp.concatenate([h3[:, :hlf, :] + h3[:, hlf:2 * hlf, :],
                             h3[:, 2 * hlf:, :]], axis=1))
        k2 = k2 - hlf
    o_ref[...] = h3.reshape(nsub, h.shape[1]) * (1.0 / ksub)


def _mlp_pool(h0, agg, w1, b1, w2, b2, ksub):
    bk, hdim = h0.shape
    blk = 12800
    nsub_blk = blk // ksub
    return pl.pallas_call(
        functools.partial(_mlp_pool_body, ksub),
        grid=(bk // blk,),
        in_specs=[
            pl.BlockSpec((blk, hdim), lambda i: (i, 0)),
            pl.BlockSpec((blk, hdim), lambda i: (i, 0)),
            pl.BlockSpec((hdim, hdim), lambda i: (0, 0)),
            pl.BlockSpec((1, hdim), lambda i: (0, 0)),
            pl.BlockSpec((hdim, hdim), lambda i: (0, 0)),
            pl.BlockSpec((1, hdim), lambda i: (0, 0)),
        ],
        out_specs=pl.BlockSpec((nsub_blk, hdim), lambda i: (i, 0)),
        out_shape=jax.ShapeDtypeStruct((bk // ksub, hdim), jnp.float32),
    )(h0, agg, w1, b1.reshape(1, -1), w2, b2.reshape(1, -1))


def _head_body(num_classes, gid_ref, reps_ref, wh1_ref, bh1_ref,
               wh2_ref, bh2_ref, logits_ref, probs_ref, preds_ref,
               onehot_ref, acc, cnt):
    i = pl.program_id(0)
    ngraphs = acc.shape[0]
    nblk = reps_ref.shape[0]

    @pl.when(i == 0)
    def _():
        acc[...] = jnp.zeros_like(acc)
        cnt[...] = jnp.zeros_like(cnt)

    gid = gid_ref[0, 0, :]
    gio = jax.lax.broadcasted_iota(jnp.int32, (ngraphs, nblk), 0)
    q = (gid[None, :] == gio).astype(jnp.float32)
    acc[...] += jnp.dot(q, reps_ref[...], preferred_element_type=jnp.float32,
                        precision=lax.Precision.HIGHEST)
    cnt[...] += jnp.broadcast_to(jnp.sum(q, axis=1, keepdims=True),
                                 cnt.shape)

    @pl.when(i == pl.num_programs(0) - 1)
    def _():
        per_graph = acc[...] / jnp.maximum(cnt[...], 1.0)
        t = jnp.maximum(
            jnp.dot(per_graph, wh1_ref[...], preferred_element_type=jnp.float32)
            + bh1_ref[...], 0.0)
        lg = jnp.dot(t, wh2_ref[...], preferred_element_type=jnp.float32) \
            + bh2_ref[...]
        col = jax.lax.broadcasted_iota(jnp.int32, lg.shape, 1)
        valid = col < num_classes
        lm = jnp.max(jnp.where(valid, lg, -jnp.inf), axis=1, keepdims=True)
        e = jnp.where(valid, jnp.exp(lg - lm), 0.0)
        probs = e / jnp.sum(e, axis=1, keepdims=True)
        pred = jnp.min(jnp.where((lg == lm) & valid, col, lg.shape[1]),
                       axis=1, keepdims=True)
        logits_ref[...] = lg
        probs_ref[...] = probs
        preds_ref[...] = jnp.broadcast_to(pred, preds_ref.shape)
        onehot_ref[...] = (col == pred).astype(jnp.float32)


def _head(graph_id, reps, wh1, bh1, wh2, bh2, ngraphs, num_classes):
    b, hdim = reps.shape
    blk = 2000
    nsteps = b // blk
    gid3 = graph_id.reshape(nsteps, 1, blk)
    wh2p = jnp.pad(wh2, ((0, 0), (0, hdim - num_classes)))
    bh2p = jnp.pad(bh2, (0, hdim - num_classes)).reshape(1, -1)
    out_sds = jax.ShapeDtypeStruct((ngraphs, hdim), jnp.float32)
    return pl.pallas_call(
        functools.partial(_head_body, num_classes),
        grid=(nsteps,),
        in_specs=[
            pl.BlockSpec((1, 1, blk), lambda i: (i, 0, 0)),
            pl.BlockSpec((blk, hdim), lambda i: (i, 0)),
            pl.BlockSpec((hdim, hdim), lambda i: (0, 0)),
            pl.BlockSpec((1, hdim), lambda i: (0, 0)),
            pl.BlockSpec((hdim, hdim), lambda i: (0, 0)),
            pl.BlockSpec((1, hdim), lambda i: (0, 0)),
        ],
        out_specs=[
            pl.BlockSpec((ngraphs, hdim), lambda i: (0, 0)),
            pl.BlockSpec((ngraphs, hdim), lambda i: (0, 0)),
            pl.BlockSpec((ngraphs, hdim), lambda i: (0, 0)),
            pl.BlockSpec((ngraphs, hdim), lambda i: (0, 0)),
        ],
        out_shape=[out_sds, out_sds,
                   jax.ShapeDtypeStruct((ngraphs, hdim), jnp.int32), out_sds],
        scratch_shapes=[pltpu.VMEM((ngraphs, hdim), jnp.float32),
                        pltpu.VMEM((ngraphs, hdim), jnp.float32)],
    )(gid3, reps, wh1, bh1.reshape(1, -1), wh2p, bh2p)


def kernel(x_global, nodes_t, edge_index_t, edge_ptr_t, graph_id_t, k,
           W_in, W1, b1, W2, b2, Wh1, bh1, Wh2, bh2):
    b, ksub = nodes_t.shape
    bk = b * ksub
    num_classes = Wh2.shape[1]
    ngraphs = 512
    e_total = edge_index_t.shape[1]

    xw = _matmul(x_global, W_in, 1000)
    h0 = _sc_gather(xw, nodes_t.reshape(-1))

    e_pad = ((e_total + 16 * _E_CHUNK - 1) // (16 * _E_CHUNK)) * (16 * _E_CHUNK)
    src_p = jnp.concatenate(
        [edge_index_t[0], jnp.zeros((e_pad - e_total,), jnp.int32)])
    dst_p = jnp.concatenate(
        [edge_index_t[1], jnp.full((e_pad - e_total,), bk, jnp.int32)])
    agg = _sc_scatter_add(src_p, dst_p, h0, bk)

    reps = _mlp_pool(h0, agg, W1, b1, W2, b2, ksub)

    logits_p, probs_p, preds_p, onehot_p = _head(
        graph_id_t, reps, Wh1, bh1, Wh2, bh2, ngraphs, num_classes)
    logits = logits_p[:, :num_classes]
    probs = probs_p[:, :num_classes]
    preds = preds_p[:, 0]
    one_hot = onehot_p[:, :num_classes]
    return (logits, probs, preds, one_hot)


# speedup vs baseline: 1.5485x; 1.0374x over previous
import functools

import jax
import jax.numpy as jnp
from jax import lax
from jax.experimental import pallas as pl
from jax.experimental.pallas import tpu as pltpu
from jax.experimental.pallas import tpu_sc as plsc

N_TC_WORKERS = 32


def _mm_body(x_ref, w_ref, o_ref):
    o_ref[...] = jnp.dot(x_ref[...], w_ref[...],
                         preferred_element_type=jnp.float32)


def _matmul(x, w, blk):
    n, kdim = x.shape
    m = w.shape[1]
    return pl.pallas_call(
        _mm_body,
        grid=(n // blk,),
        in_specs=[
            pl.BlockSpec((blk, kdim), lambda i: (i, 0)),
            pl.BlockSpec((kdim, m), lambda i: (0, 0)),
        ],
        out_specs=pl.BlockSpec((blk, m), lambda i: (i, 0)),
        out_shape=jax.ShapeDtypeStruct((n, m), jnp.float32),
    )(x, w)


def _sc_gather(table, idx):
    bsz = idx.shape[0]
    d = table.shape[1]
    per_w = bsz // N_TC_WORKERS
    chunk = 80
    n_chunks = per_w // chunk
    assert per_w % chunk == 0
    mesh = plsc.VectorSubcoreMesh(core_axis_name="c", subcore_axis_name="s")

    @functools.partial(
        pl.kernel, mesh=mesh,
        out_type=jax.ShapeDtypeStruct((bsz, d), jnp.float32),
        scratch_types=[
            pltpu.VMEM((chunk,), jnp.int32),
            pltpu.VMEM((chunk, d), jnp.float32),
            pltpu.SemaphoreType.DMA,
        ],
    )
    def gather_k(table_hbm, idx_hbm, out_hbm, idx_v, rows_v, sem):
        wid = lax.axis_index("s") * 2 + lax.axis_index("c")
        base = wid * per_w

        def body(i, _):
            off = base + i * chunk
            pltpu.sync_copy(idx_hbm.at[pl.ds(off, chunk)], idx_v)
            pltpu.async_copy(table_hbm.at[idx_v], rows_v, sem).wait()
            pltpu.sync_copy(rows_v, out_hbm.at[pl.ds(off, chunk)])
            return 0

        lax.fori_loop(0, n_chunks, body, 0)

    return gather_k(table, idx)


_E_CHUNK = 2048
_BATCH = 128
_PEND = 2304
_Q = 1280
_REG = 2304


def _sc_scatter_add(src_pad, dst_pad, h0, n_rows):
    d = h0.shape[1]
    e_pad = src_pad.shape[0]
    per_tile_e = e_pad // 16
    n_chunks = per_tile_e // _E_CHUNK
    assert per_tile_e % _E_CHUNK == 0
    half = n_rows // 2
    rng = 10000
    n_pass = half // rng
    assert half % n_pass == 0
    rows_per_tile = 624
    base_pieces = [(0, 128), (128, 128), (256, 128), (384, 128), (512, 112)]
    extra_piece = (15 * 624, 16)
    mesh = plsc.VectorSubcoreMesh(core_axis_name="c", subcore_axis_name="s")

    @functools.partial(
        pl.kernel, mesh=mesh,
        out_type=jax.ShapeDtypeStruct((n_rows, d), jnp.float32),
        scratch_types=[
            pltpu.VMEM((_E_CHUNK,), jnp.int32),
            pltpu.VMEM((_E_CHUNK,), jnp.int32),
            pltpu.VMEM((_PEND,), jnp.int32),
            pltpu.VMEM((_PEND,), jnp.int32),
            pltpu.VMEM((_Q,), jnp.int32),
            pltpu.VMEM((_Q,), jnp.int32),
            pltpu.VMEM((_BATCH,), jnp.int32),
        ] + [pltpu.VMEM((16,), jnp.int32) for _ in range(8)] + [
            pltpu.VMEM((16,), jnp.int32),
            pltpu.VMEM((4096,), jnp.int32),
            pltpu.VMEM((4096,), jnp.int32),
            pltpu.VMEM((256,), jnp.int32),
            pltpu.VMEM((256,), jnp.int32),
            pltpu.VMEM((256,), jnp.int32),
            pltpu.VMEM((_BATCH, 128), jnp.float32),
            pltpu.VMEM((656, 128), jnp.float32),
            pltpu.VMEM_SHARED((16 * _REG,), jnp.int32),
            pltpu.VMEM_SHARED((16 * _REG,), jnp.int32),
            pltpu.VMEM_SHARED((256,), jnp.int32),
            pltpu.SemaphoreType.DMA,
        ],
    )
    def scatter_k(src_hbm, dst_hbm, h0_hbm, zeros_hbm, agg_hbm,
                  src_v, dst_v, pend_src, pend_dst, q_src, q_dst, bat_src,
                  b16_0, b16_1, b16_2, b16_3, b16_4, b16_5, b16_6, b16_7,
                  cnt_pub, sbuf_src, sbuf_dst, sbuf_cnt, ovf_src, ovf_dst,
                  rows_v, acc, stage_src, stage_dst, stage_cnt, sem):
        bat16 = [b16_0, b16_1, b16_2, b16_3, b16_4, b16_5, b16_6, b16_7]
        c = lax.axis_index("c")
        s = lax.axis_index("s")

        lane = lax.iota(jnp.int32, 16)
        one = jnp.full((16,), 1, jnp.int32)
        zero16 = jnp.full((16,), 0, jnp.int32)
        rng_u = jnp.full((16,), rng, jnp.uint32)
        dummy_v = jnp.full((16,), rng, jnp.int32)
        lane_u = lane.astype(jnp.uint32)
        pshift = [(jnp.maximum(lane - kk, 0),
                   lane_u >= jnp.full((16,), kk, jnp.uint32))
                  for kk in (1, 2, 4, 8)]
        rot_idx = [jnp.maximum(lane - r, 0) for r in range(1, 16)]
        tgt = lane + one
        own_lo = s * rows_per_tile
        own_sz = jnp.where(s == 15, 640, 624)
        own_lo_v = jnp.full((16,), own_lo, jnp.int32)
        own_sz_u = jnp.full((16,), own_sz, jnp.int32).astype(jnp.uint32)

        def _id(x):
            return x

        def init_pend(i, _):
            pend_src[pl.ds(i * 16, 16)] = zero16
            return 0
        lax.fori_loop(0, _PEND // 16, init_pend, 0)

        def init_q(i, _):
            q_src[pl.ds(i * 16, 16)] = zero16
            return 0
        lax.fori_loop(0, _Q // 16, init_q, 0)

        def append_compact(dref, sref, dvx, svx, mind, cur):
            p = mind
            for idxk, mk in pshift:
                p = p + jnp.where(mk, jnp.take(p, idxk), zero16)
            cnt = p[15]

            @pl.when(cnt > 0)
            def _():
                pos = zero16
                for kk in (8, 4, 2, 1):
                    npos = pos + jnp.full((16,), kk, jnp.int32)
                    pv = jnp.take(p, npos - one)
                    pos = jnp.where(pv < tgt, npos, pos)
                dref[pl.ds(cur, 16)] = jnp.take(dvx, pos)
                sref[pl.ds(cur, 16)] = jnp.take(svx, pos)
            return cur + cnt

        def drain_batch(cur):
            cur = cur - _BATCH
            for j in range(8):
                bat_src[pl.ds(j * 16, 16)] = q_src[pl.ds(cur + j * 16, 16)]
            pltpu.async_copy(h0_hbm.at[bat_src], rows_v, sem).wait()

            def addrow(j, _):
                dvb = q_dst[pl.ds(cur + j * 16, 16)]
                for l in range(16):
                    dloc = dvb[l]
                    for u in range(8):
                        acc[dloc, pl.ds(u * 16, 16)] = (
                            acc[dloc, pl.ds(u * 16, 16)]
                            + rows_v[j * 16 + l, pl.ds(u * 16, 16)])
                return 0
            lax.fori_loop(0, 8, addrow, 0)
            return cur

        qdummy_v = jnp.full((16,), 648, jnp.int32)

        def drain_all(cur):
            def pad_step(_, cu):
                def do_pad(c2):
                    q_dst[pl.ds(c2, 16)] = qdummy_v
                    return c2 + jnp.minimum(16, _BATCH - (c2 & (_BATCH - 1)))
                return lax.cond((cu & (_BATCH - 1)) != 0, do_pad, _id, cu)

            def round_(_, cu):
                def do_round(c2):
                    c2 = lax.fori_loop(0, 8, pad_step, c2)
                    return drain_batch(c2)
                return lax.cond(cu > 0, do_round, _id, cu)

            return lax.fori_loop(0, 4, round_, cur)

        def do_pass(p, _):
            lo = c * half + p * rng
            zcs = [pltpu.async_copy(zeros_hbm, acc.at[pl.ds(zoff, 128)], sem)
                   for zoff in (0, 128, 256, 384, 512)]
            zcs.append(pltpu.async_copy(zeros_hbm.at[pl.ds(0, 16)],
                                        acc.at[pl.ds(640, 16)], sem))
            for zc in zcs:
                zc.wait()
            plsc.subcore_barrier()

            lo_v = jnp.full((16,), lo, jnp.int32)

            def do_chunk(ci, qcur):
                ebase = s * per_tile_e + ci * _E_CHUNK
                ec1 = pltpu.async_copy(src_hbm.at[pl.ds(ebase, _E_CHUNK)],
                                       src_v, sem)
                ec2 = pltpu.async_copy(dst_hbm.at[pl.ds(ebase, _E_CHUNK)],
                                       dst_v, sem)
                ec1.wait()
                ec2.wait()

                def scan(i, cur):
                    dv = dst_v[pl.ds(i * 16, 16)]
                    sv = src_v[pl.ds(i * 16, 16)]
                    m = (dv - lo_v).astype(jnp.uint32) < rng_u
                    mi = jnp.where(m, one, zero16)
                    return append_compact(pend_dst, pend_src,
                                          dv - lo_v, sv, mi, cur)

                cur = lax.fori_loop(0, _E_CHUNK // 16, scan, 0)

                def padp(c2):
                    pend_dst[pl.ds(c2, 16)] = dummy_v
                    return c2 + (16 - (c2 & 15))
                cur = lax.cond((cur & 15) != 0, padp, _id, cur)

                cnt_pub[...] = jnp.full((16,), cur, jnp.int32)
                pltpu.sync_copy(cnt_pub, stage_cnt.at[pl.ds(s * 16, 16)])
                nb = (cur + 255) // 256

                def cpb(b, _):
                    k1 = pltpu.async_copy(
                        pend_src.at[pl.ds(b * 256, 256)],
                        stage_src.at[pl.ds(s * _REG + b * 256, 256)], sem)
                    k2 = pltpu.async_copy(
                        pend_dst.at[pl.ds(b * 256, 256)],
                        stage_dst.at[pl.ds(s * _REG + b * 256, 256)], sem)
                    k1.wait()
                    k2.wait()
                    return 0
                lax.fori_loop(0, nb, cpb, 0)
                plsc.subcore_barrier()

                pltpu.sync_copy(stage_cnt, sbuf_cnt)
                cps = []
                for pp in range(16):
                    cps.append(pltpu.async_copy(
                        stage_src.at[pl.ds(pp * _REG, 256)],
                        sbuf_src.at[pl.ds(pp * 256, 256)], sem))
                    cps.append(pltpu.async_copy(
                        stage_dst.at[pl.ds(pp * _REG, 256)],
                        sbuf_dst.at[pl.ds(pp * 256, 256)], sem))
                for cp in cps:
                    cp.wait()

                def scan_blk(dref, sref, boff, cnt_p, base_g, qcur):
                    cnt_u = jnp.full((16,), cnt_p, jnp.int32).astype(jnp.uint32)
                    inblk = jnp.clip(cnt_p - base_g, 0, 256)
                    nv = (inblk + 15) // 16

                    def per_vreg(v, qcur):
                        dv = dref[pl.ds(boff + v * 16, 16)]
                        sv = sref[pl.ds(boff + v * 16, 16)]
                        gidx = jnp.full((16,), base_g, jnp.int32) \
                            + jnp.full((16,), v * 16, jnp.int32) + lane
                        okv = gidx.astype(jnp.uint32) < cnt_u
                        mine = (dv - own_lo_v).astype(jnp.uint32) < own_sz_u
                        ind = jnp.where(mine, jnp.where(okv, one, zero16),
                                        zero16)
                        qcur = append_compact(q_dst, q_src,
                                              dv - own_lo_v, sv, ind, qcur)
                        qcur = lax.cond(qcur >= 192, drain_batch, _id, qcur)
                        qcur = lax.cond(qcur >= 192, drain_batch, _id, qcur)
                        return qcur

                    return lax.fori_loop(0, nv, per_vreg, qcur)

                def per_prod(pp, qcur):
                    cnt_p = sbuf_cnt[pl.ds(pp * 16, 16)][0]
                    qcur = scan_blk(sbuf_dst, sbuf_src, pp * 256, cnt_p,
                                    0, qcur)
                    nb2 = (cnt_p + 255) // 256

                    def per_blk(b, qcur):
                        pltpu.sync_copy(
                            stage_src.at[pl.ds(pp * _REG + b * 256, 256)],
                            ovf_src)
                        pltpu.sync_copy(
                            stage_dst.at[pl.ds(pp * _REG + b * 256, 256)],
                            ovf_dst)
                        return scan_blk(ovf_dst, ovf_src, 0, cnt_p,
                                        b * 256, qcur)

                    return lax.cond(
                        nb2 > 1,
                        lambda qc: lax.fori_loop(1, nb2, per_blk, qc),
                        _id, qcur)

                qcur = lax.fori_loop(0, 16, per_prod, qcur)
                plsc.subcore_barrier()
                return qcur

            qcur = lax.fori_loop(0, n_chunks, do_chunk, 0)
            drain_all(qcur)
            wcs = [pltpu.async_copy(acc.at[pl.ds(off, sz)],
                                    agg_hbm.at[pl.ds(lo + own_lo + off, sz)],
                                    sem)
                   for off, sz in base_pieces]
            for wc in wcs:
                wc.wait()

            @pl.when(s == 15)
            def _():
                pltpu.sync_copy(acc.at[pl.ds(624, 16)],
                                agg_hbm.at[pl.ds(lo + 9984, 16)])
            return 0

        lax.fori_loop(0, n_pass, do_pass, 0)

    zeros_in = jnp.zeros((128, 128), jnp.float32)
    return scatter_k(src_pad, dst_pad, h0, zeros_in)


def _mlp_pool_body(ksub, h0_ref, agg_ref, w1_ref, b1_ref, w2_ref, b2_ref, o_ref):
    h = h0_ref[...] + agg_ref[...]
    h = jnp.maximum(jnp.dot(h, w1_ref[...], preferred_element_type=jnp.float32)
                    + b1_ref[...], 0.0)
    h = jnp.maximum(jnp.dot(h, w2_ref[...], preferred_element_type=jnp.float32)
                    + b2_ref[...], 0.0)
    nsub = h.shape[0] // ksub
    h3 = h.reshape(nsub, ksub, h.shape[1])
    k2 = ksub
    while k2 > 1:
        hlf = k2 // 2
        h3 = h3[:, :hlf, :] + h3[:, hlf:2 * hlf, :] if k2 % 2 == 0 else (
            jnp.concatenate([h3[:, :hlf, :] + h3[:, hlf:2 * hlf, :],
                             h3[:, 2 * hlf:, :]], axis=1))
        k2 = k2 - hlf
    o_ref[...] = h3.reshape(nsub, h.shape[1]) * (1.0 / ksub)


def _mlp_pool(h0, agg, w1, b1, w2, b2, ksub):
    bk, hdim = h0.shape
    blk = 12800
    nsub_blk = blk // ksub
    return pl.pallas_call(
        functools.partial(_mlp_pool_body, ksub),
        grid=(bk // blk,),
        in_specs=[
            pl.BlockSpec((blk, hdim), lambda i: (i, 0)),
            pl.BlockSpec((blk, hdim), lambda i: (i, 0)),
            pl.BlockSpec((hdim, hdim), lambda i: (0, 0)),
            pl.BlockSpec((1, hdim), lambda i: (0, 0)),
            pl.BlockSpec((hdim, hdim), lambda i: (0, 0)),
            pl.BlockSpec((1, hdim), lambda i: (0, 0)),
        ],
        out_specs=pl.BlockSpec((nsub_blk, hdim), lambda i: (i, 0)),
        out_shape=jax.ShapeDtypeStruct((bk // ksub, hdim), jnp.float32),
    )(h0, agg, w1, b1.reshape(1, -1), w2, b2.reshape(1, -1))


def _head_body(num_classes, gid_ref, reps_ref, wh1_ref, bh1_ref,
               wh2_ref, bh2_ref, logits_ref, probs_ref, preds_ref,
               onehot_ref, acc, cnt):
    i = pl.program_id(0)
    ngraphs = acc.shape[0]
    nblk = reps_ref.shape[0]

    @pl.when(i == 0)
    def _():
        acc[...] = jnp.zeros_like(acc)
        cnt[...] = jnp.zeros_like(cnt)

    gid = gid_ref[0, 0, :]
    gio = jax.lax.broadcasted_iota(jnp.int32, (ngraphs, nblk), 0)
    q = (gid[None, :] == gio).astype(jnp.float32)
    acc[...] += jnp.dot(q, reps_ref[...], preferred_element_type=jnp.float32,
                        precision=lax.Precision.HIGHEST)
    cnt[...] += jnp.broadcast_to(jnp.sum(q, axis=1, keepdims=True),
                                 cnt.shape)

    @pl.when(i == pl.num_programs(0) - 1)
    def _():
        per_graph = acc[...] / jnp.maximum(cnt[...], 1.0)
        t = jnp.maximum(
            jnp.dot(per_graph, wh1_ref[...], preferred_element_type=jnp.float32)
            + bh1_ref[...], 0.0)
        lg = jnp.dot(t, wh2_ref[...], preferred_element_type=jnp.float32) \
            + bh2_ref[...]
        col = jax.lax.broadcasted_iota(jnp.int32, lg.shape, 1)
        valid = col < num_classes
        lm = jnp.max(jnp.where(valid, lg, -jnp.inf), axis=1, keepdims=True)
        e = jnp.where(valid, jnp.exp(lg - lm), 0.0)
        probs = e / jnp.sum(e, axis=1, keepdims=True)
        pred = jnp.min(jnp.where((lg == lm) & valid, col, lg.shape[1]),
                       axis=1, keepdims=True)
        logits_ref[...] = lg
        probs_ref[...] = probs
        preds_ref[...] = jnp.broadcast_to(pred, preds_ref.shape)
        onehot_ref[...] = (col == pred).astype(jnp.float32)


def _head(graph_id, reps, wh1, bh1, wh2, bh2, ngraphs, num_classes):
    b, hdim = reps.shape
    blk = 2000
    nsteps = b // blk
    gid3 = graph_id.reshape(nsteps, 1, blk)
    wh2p = jnp.pad(wh2, ((0, 0), (0, hdim - num_classes)))
    bh2p = jnp.pad(bh2, (0, hdim - num_classes)).reshape(1, -1)
    out_sds = jax.ShapeDtypeStruct((ngraphs, hdim), jnp.float32)
    return pl.pallas_call(
        functools.partial(_head_body, num_classes),
        grid=(nsteps,),
        in_specs=[
            pl.BlockSpec((1, 1, blk), lambda i: (i, 0, 0)),
            pl.BlockSpec((blk, hdim), lambda i: (i, 0)),
            pl.BlockSpec((hdim, hdim), lambda i: (0, 0)),
            pl.BlockSpec((1, hdim), lambda i: (0, 0)),
            pl.BlockSpec((hdim, hdim), lambda i: (0, 0)),
            pl.BlockSpec((1, hdim), lambda i: (0, 0)),
        ],
        out_specs=[
            pl.BlockSpec((ngraphs, hdim), lambda i: (0, 0)),
            pl.BlockSpec((ngraphs, hdim), lambda i: (0, 0)),
            pl.BlockSpec((ngraphs, hdim), lambda i: (0, 0)),
            pl.BlockSpec((ngraphs, hdim), lambda i: (0, 0)),
        ],
        out_shape=[out_sds, out_sds,
                   jax.ShapeDtypeStruct((ngraphs, hdim), jnp.int32), out_sds],
        scratch_shapes=[pltpu.VMEM((ngraphs, hdim), jnp.float32),
                        pltpu.VMEM((ngraphs, hdim), jnp.float32)],
    )(gid3, reps, wh1, bh1.reshape(1, -1), wh2p, bh2p)


def kernel(x_global, nodes_t, edge_index_t, edge_ptr_t, graph_id_t, k,
           W_in, W1, b1, W2, b2, Wh1, bh1, Wh2, bh2):
    b, ksub = nodes_t.shape
    bk = b * ksub
    num_classes = Wh2.shape[1]
    ngraphs = 512
    e_total = edge_index_t.shape[1]

    xw = _matmul(x_global, W_in, 1000)
    h0 = _sc_gather(xw, nodes_t.reshape(-1))

    e_pad = ((e_total + 16 * _E_CHUNK - 1) // (16 * _E_CHUNK)) * (16 * _E_CHUNK)
    src_p = jnp.concatenate(
        [edge_index_t[0], jnp.zeros((e_pad - e_total,), jnp.int32)])
    dst_p = jnp.concatenate(
        [edge_index_t[1], jnp.full((e_pad - e_total,), bk, jnp.int32)])
    agg = _sc_scatter_add(src_p, dst_p, h0, bk)

    reps = _mlp_pool(h0, agg, W1, b1, W2, b2, ksub)

    logits_p, probs_p, preds_p, onehot_p = _head(
        graph_id_t, reps, Wh1, bh1, Wh2, bh2, ngraphs, num_classes)
    logits = logits_p[:, :num_classes]
    probs = probs_p[:, :num_classes]
    preds = preds_p[:, 0]
    one_hot = onehot_p[:, :num_classes]
    return (logits, probs, preds, one_hot)
